# baseline (speedup 1.0000x reference)
import functools

import jax
import jax.numpy as jnp
from jax import lax
from jax.experimental import pallas as pl
from jax.experimental.pallas import tpu as pltpu
from jax.experimental.pallas import tpu_sc as plsc

NW = 32
P_CHUNK = 8


def _emb_kernel(L, B, V, D):
    R = L * B
    rows_per_w = R // NW
    pos_per_w = L // NW
    chunks = pos_per_w // P_CHUNK
    rows_chunk = B * P_CHUNK

    mesh = plsc.VectorSubcoreMesh(core_axis_name="c", subcore_axis_name="s")

    @functools.partial(
        pl.kernel,
        mesh=mesh,
        out_type=jax.ShapeDtypeStruct((R, D), jnp.float32),
        scratch_types=[
            pltpu.VMEM((rows_per_w,), jnp.int32),
            pltpu.VMEM((rows_chunk, D), jnp.float32),
            pltpu.VMEM((P_CHUNK, D), jnp.float32),
            pltpu.SemaphoreType.DMA,
        ],
    )
    def run(x_hbm, w_hbm, pe_hbm, out_hbm, idx_v, rows_v, pe_v, sem):
        wid = lax.axis_index("s") * 2 + lax.axis_index("c")
        row_base = wid * rows_per_w
        pos_base = wid * pos_per_w
        pltpu.sync_copy(x_hbm.at[pl.ds(row_base, rows_per_w)], idx_v)

        def chunk_body(g, carry):
            row0 = row_base + g * rows_chunk
            pos0 = pos_base + g * P_CHUNK
            pltpu.sync_copy(pe_hbm.at[pl.ds(pos0, P_CHUNK)], pe_v)
            gat = pltpu.async_copy(
                w_hbm.at[idx_v.at[pl.ds(g * rows_chunk, rows_chunk)]],
                rows_v, sem)
            gat.wait()

            def p_body(p, c2):
                def c_body(c, c3):
                    col = c * 16
                    pev = pe_v[p, pl.ds(col, 16)]
                    for b in range(B):
                        r = B * p + b
                        rows_v[r, pl.ds(col, 16)] = (
                            rows_v[r, pl.ds(col, 16)] + pev)
                    return c3
                return lax.fori_loop(0, D // 16, c_body, c2)

            lax.fori_loop(0, P_CHUNK, p_body, 0)
            pltpu.sync_copy(rows_v, out_hbm.at[pl.ds(row0, rows_chunk)])
            return carry

        lax.fori_loop(0, chunks, chunk_body, 0)

    return run


def kernel(x, W_emb, pe):
    L, B = x.shape
    V, D = W_emb.shape
    x_flat = x.reshape(L * B)
    pe_flat = pe.reshape(pe.shape[0], D)
    out = _emb_kernel(L, B, V, D)(x_flat, W_emb, pe_flat)
    return out.reshape(L, B, D)


# speedup vs baseline: 1.0398x; 1.0398x over previous
import functools

import jax
import jax.numpy as jnp
from jax import lax
from jax.experimental import pallas as pl
from jax.experimental.pallas import tpu as pltpu
from jax.experimental.pallas import tpu_sc as plsc

NC = 2
NS = 16
NW = NC * NS
P_CHUNK = 4


def _emb_kernel(L, B, V, D):
    R = L * B
    rows_per_w = R // NW
    pos_per_w = L // NW
    chunks = pos_per_w // P_CHUNK
    rows_chunk = B * P_CHUNK

    mesh = plsc.VectorSubcoreMesh(core_axis_name="c", subcore_axis_name="s")

    @functools.partial(
        pl.kernel,
        mesh=mesh,
        out_type=jax.ShapeDtypeStruct((R, D), jnp.float32),
        scratch_types=[
            pltpu.VMEM((rows_per_w,), jnp.int32),
            [pltpu.VMEM((rows_chunk, D), jnp.float32) for _ in range(2)],
            [pltpu.VMEM((rows_chunk, D), jnp.float32) for _ in range(2)],
            [pltpu.VMEM((P_CHUNK, D), jnp.float32) for _ in range(2)],
            [pltpu.SemaphoreType.DMA for _ in range(6)],
        ],
    )
    def run(x_hbm, w_hbm, pe_hbm, out_hbm, idx_v, gbuf, obuf, pebuf, sems):
        gsem = sems[0:2]
        psem = sems[2:4]
        osem = sems[4:6]
        sid = lax.axis_index("s")
        wid = sid * NC + lax.axis_index("c")
        row_base = wid * rows_per_w
        pos_base = wid * pos_per_w
        pltpu.sync_copy(x_hbm.at[pl.ds(row_base, rows_per_w)], idx_v)

        def issue(gg, b):
            pltpu.async_copy(
                w_hbm.at[idx_v.at[pl.ds(gg * rows_chunk, rows_chunk)]],
                gbuf[b], gsem[b])
            pltpu.async_copy(
                pe_hbm.at[pl.ds(pos_base + gg * P_CHUNK, P_CHUNK)],
                pebuf[b], psem[b])

        issue(0, 0)
        issue(1, 1)

        def slot(gg, b):
            row0 = row_base + gg * rows_chunk
            pltpu.make_async_copy(
                w_hbm.at[idx_v.at[pl.ds(gg * rows_chunk, rows_chunk)]],
                gbuf[b], gsem[b]).wait()
            pltpu.make_async_copy(
                pe_hbm.at[pl.ds(pos_base + gg * P_CHUNK, P_CHUNK)],
                pebuf[b], psem[b]).wait()
            @pl.when(gg >= 2)
            def _():
                pltpu.make_async_copy(
                    obuf[b],
                    out_hbm.at[pl.ds(row0 - 2 * rows_chunk, rows_chunk)],
                    osem[b]).wait()

            @plsc.parallel_loop(0, P_CHUNK * (D // 16))
            def add_body(i):
                p = lax.shift_right_logical(i, 6)
                col = (i & 63) * 16
                pev = pebuf[b][p, pl.ds(col, 16)]
                for bb in range(B):
                    r = B * p + bb
                    obuf[b][r, pl.ds(col, 16)] = (
                        gbuf[b][r, pl.ds(col, 16)] + pev)

            pltpu.async_copy(obuf[b], out_hbm.at[pl.ds(row0, rows_chunk)],
                             osem[b])

            @pl.when(gg + 2 < chunks)
            def _():
                issue(gg + 2, b)

        def pair_body(i, carry):
            slot(2 * i, 0)
            slot(2 * i + 1, 1)
            return carry

        lax.fori_loop(0, chunks // 2, pair_body, 0)

        for b in range(2):
            gg = chunks - 2 + b
            pltpu.make_async_copy(
                obuf[b],
                out_hbm.at[pl.ds(row_base + gg * rows_chunk, rows_chunk)],
                osem[b]).wait()

    return run


def kernel(x, W_emb, pe):
    L, B = x.shape
    V, D = W_emb.shape
    x_flat = x.reshape(L * B)
    pe_flat = pe.reshape(pe.shape[0], D)
    out = _emb_kernel(L, B, V, D)(x_flat, W_emb, pe_flat)
    return out.reshape(L, B, D)


# speedup vs baseline: 1.4506x; 1.3951x over previous
import functools

import jax
import jax.numpy as jnp
from jax import lax
from jax.experimental import pallas as pl
from jax.experimental.pallas import tpu as pltpu
from jax.experimental.pallas import tpu_sc as plsc

NC = 2
NS = 16
NW = NC * NS
P_CHUNK = 4


def _emb_kernel(L, B, V, D):
    R = L * B
    rows_per_w = R // NW
    pos_per_w = L // NW
    chunks = pos_per_w // P_CHUNK
    rows_chunk = B * P_CHUNK

    mesh = plsc.VectorSubcoreMesh(core_axis_name="c", subcore_axis_name="s")

    @functools.partial(
        pl.kernel,
        mesh=mesh,
        out_type=jax.ShapeDtypeStruct((L, B, D), jnp.float32),
        scratch_types=[
            pltpu.VMEM((rows_per_w,), jnp.int32),
            [pltpu.VMEM((rows_chunk, D), jnp.float32) for _ in range(2)],
            [pltpu.VMEM((P_CHUNK, B, D), jnp.float32) for _ in range(2)],
            [pltpu.VMEM((P_CHUNK, D), jnp.float32) for _ in range(2)],
            [pltpu.SemaphoreType.DMA for _ in range(6)],
        ],
    )
    def run(x_hbm, w_hbm, pe_hbm, out_hbm, idx_v, gbuf, obuf, pebuf, sems):
        gsem = sems[0:2]
        psem = sems[2:4]
        osem = sems[4:6]
        sid = lax.axis_index("s")
        wid = sid * NC + lax.axis_index("c")
        row_base = wid * rows_per_w
        pos_base = wid * pos_per_w
        pltpu.sync_copy(x_hbm.at[pl.ds(row_base, rows_per_w)], idx_v)

        def issue(gg, b):
            pltpu.async_copy(
                w_hbm.at[idx_v.at[pl.ds(gg * rows_chunk, rows_chunk)]],
                gbuf[b], gsem[b])
            pltpu.async_copy(
                pe_hbm.at[pl.ds(pos_base + gg * P_CHUNK, P_CHUNK)],
                pebuf[b], psem[b])

        issue(0, 0)
        issue(1, 1)

        def slot(gg, b):
            pos0 = pos_base + gg * P_CHUNK
            pltpu.make_async_copy(
                w_hbm.at[idx_v.at[pl.ds(gg * rows_chunk, rows_chunk)]],
                gbuf[b], gsem[b]).wait()
            pltpu.make_async_copy(
                pe_hbm.at[pl.ds(pos0, P_CHUNK)], pebuf[b], psem[b]).wait()
            @pl.when(gg >= 2)
            def _():
                pltpu.make_async_copy(
                    obuf[b],
                    out_hbm.at[pl.ds(pos0 - 2 * P_CHUNK, P_CHUNK)],
                    osem[b]).wait()

            def p_body(p, carry):
                @plsc.parallel_loop(0, D // 16, unroll=8)
                def add_body(c):
                    col = c * 16
                    pev = pebuf[b][p, pl.ds(col, 16)]
                    for bb in range(B):
                        obuf[b][p, bb, pl.ds(col, 16)] = (
                            gbuf[b][B * p + bb, pl.ds(col, 16)] + pev)
                return carry

            lax.fori_loop(0, P_CHUNK, p_body, 0)
            pltpu.async_copy(obuf[b], out_hbm.at[pl.ds(pos0, P_CHUNK)],
                             osem[b])

            @pl.when(gg + 2 < chunks)
            def _():
                issue(gg + 2, b)

        def pair_body(i, carry):
            slot(2 * i, 0)
            slot(2 * i + 1, 1)
            return carry

        lax.fori_loop(0, chunks // 2, pair_body, 0)

        for b in range(2):
            gg = chunks - 2 + b
            pltpu.make_async_copy(
                obuf[b],
                out_hbm.at[pl.ds(pos_base + gg * P_CHUNK, P_CHUNK)],
                osem[b]).wait()

    return run


def kernel(x, W_emb, pe):
    L, B = x.shape
    V, D = W_emb.shape
    x_flat = x.reshape(L * B)
    pe_flat = pe.reshape(pe.shape[0], D)
    return _emb_kernel(L, B, V, D)(x_flat, W_emb, pe_flat)


# speedup vs baseline: 3.4523x; 2.3799x over previous
import functools

import jax
import jax.numpy as jnp
from jax import lax
from jax.experimental import pallas as pl
from jax.experimental.pallas import tpu as pltpu
from jax.experimental.pallas import tpu_sc as plsc

NC = 2
NS = 16
NW = NC * NS
P_CHUNK = 4


def _emb_kernel(L, B, V, D):
    R = L * B
    rows_per_w = R // NW
    pos_per_w = L // NW
    chunks = pos_per_w // P_CHUNK
    rows_chunk = B * P_CHUNK

    mesh = plsc.VectorSubcoreMesh(core_axis_name="c", subcore_axis_name="s")

    @functools.partial(
        pl.kernel,
        mesh=mesh,
        out_type=jax.ShapeDtypeStruct((L, B, D), jnp.float32),
        scratch_types=[
            pltpu.VMEM((rows_per_w,), jnp.int32),
            pltpu.VMEM_SHARED((V, 8, D // 8), jnp.float32),
            [pltpu.VMEM((rows_chunk, 8, D // 8), jnp.float32)
             for _ in range(2)],
            [pltpu.VMEM((P_CHUNK, B, D), jnp.float32) for _ in range(2)],
            [pltpu.VMEM((P_CHUNK, D), jnp.float32) for _ in range(2)],
            [pltpu.SemaphoreType.DMA for _ in range(6)],
        ],
    )
    def run(x_hbm, w_hbm, pe_hbm, out_hbm, idx_v, w_s, gbuf, obuf, pebuf,
            sems):
        gsem = sems[0:2]
        psem = sems[2:4]
        osem = sems[4:6]
        sid = lax.axis_index("s")
        wid = sid * NC + lax.axis_index("c")
        row_base = wid * rows_per_w
        pos_base = wid * pos_per_w
        pltpu.sync_copy(x_hbm.at[pl.ds(row_base, rows_per_w)], idx_v)

        @pl.when(sid == 0)
        def _():
            pltpu.sync_copy(w_hbm, w_s)

        plsc.subcore_barrier()

        def issue(gg, b):
            pltpu.async_copy(
                w_s.at[idx_v.at[pl.ds(gg * rows_chunk, rows_chunk)]],
                gbuf[b], gsem[b])
            pltpu.async_copy(
                pe_hbm.at[pl.ds(pos_base + gg * P_CHUNK, P_CHUNK)],
                pebuf[b], psem[b])

        issue(0, 0)
        issue(1, 1)

        def slot(gg, b):
            pos0 = pos_base + gg * P_CHUNK
            pltpu.make_async_copy(
                w_s.at[idx_v.at[pl.ds(gg * rows_chunk, rows_chunk)]],
                gbuf[b], gsem[b]).wait()
            pltpu.make_async_copy(
                pe_hbm.at[pl.ds(pos0, P_CHUNK)], pebuf[b], psem[b]).wait()
            @pl.when(gg >= 2)
            def _():
                pltpu.make_async_copy(
                    obuf[b],
                    out_hbm.at[pl.ds(pos0 - 2 * P_CHUNK, P_CHUNK)],
                    osem[b]).wait()

            def p_body(p, carry):
                @plsc.parallel_loop(0, D // 16, unroll=8)
                def add_body(c):
                    col = c * 16
                    s8 = lax.shift_right_logical(c, 3)
                    off = (c & 7) * 16
                    pev = pebuf[b][p, pl.ds(col, 16)]
                    for bb in range(B):
                        obuf[b][p, bb, pl.ds(col, 16)] = (
                            gbuf[b][B * p + bb, s8, pl.ds(off, 16)] + pev)
                return carry

            lax.fori_loop(0, P_CHUNK, p_body, 0)
            pltpu.async_copy(obuf[b], out_hbm.at[pl.ds(pos0, P_CHUNK)],
                             osem[b])

            @pl.when(gg + 2 < chunks)
            def _():
                issue(gg + 2, b)

        def pair_body(i, carry):
            slot(2 * i, 0)
            slot(2 * i + 1, 1)
            return carry

        lax.fori_loop(0, chunks // 2, pair_body, 0)

        for b in range(2):
            gg = chunks - 2 + b
            pltpu.make_async_copy(
                obuf[b],
                out_hbm.at[pl.ds(pos_base + gg * P_CHUNK, P_CHUNK)],
                osem[b]).wait()

    return run


def kernel(x, W_emb, pe):
    L, B = x.shape
    V, D = W_emb.shape
    x_flat = x.reshape(L * B)
    w3 = W_emb.reshape(V, 8, D // 8)
    pe_flat = pe.reshape(pe.shape[0], D)
    return _emb_kernel(L, B, V, D)(x_flat, w3, pe_flat)
